# baseline (speedup 1.0000x reference)
import functools

import jax
import jax.numpy as jnp
from jax import lax
from jax.experimental import pallas as pl
from jax.experimental.pallas import tpu as pltpu
from jax.experimental.pallas import tpu_sc as plsc

_B = 16384
_D = 128
_NUM = 1000000
_R = 4

_NT = 16
_TPT = _B // _NT
_CH = 128
_NCH = _TPT // _CH

_NW = 32
_RPW = _B // _NW
_GCH = _RPW // _CH


def _ln_body(x_ref, g_ref, b_ref, o_ref):
    x = x_ref[...]
    mu = jnp.mean(x, axis=-1, keepdims=True)
    xc = x - mu
    var = jnp.mean(xc * xc, axis=-1, keepdims=True)
    o_ref[...] = xc * lax.rsqrt(var + 1e-5) * g_ref[...] + b_ref[...]


def _layer_norm_tc(x, g, b):
    blk = 1024
    return pl.pallas_call(
        _ln_body,
        grid=(_B // blk,),
        in_specs=[
            pl.BlockSpec((blk, _D), lambda i: (i, 0)),
            pl.BlockSpec((1, _D), lambda i: (0, 0)),
            pl.BlockSpec((1, _D), lambda i: (0, 0)),
        ],
        out_specs=pl.BlockSpec((blk, _D), lambda i: (i, 0)),
        out_shape=jax.ShapeDtypeStruct((_B, _D), jnp.float32),
    )(x, g.reshape(1, _D), b.reshape(1, _D))


def _fused_body(ids_hbm, upd_hbm, out_hbm,
                tbl, wsh, ids_v, val_v, w_v, idx_v, gidx_v, rows_v, sem,
                sem_g):
    c = lax.axis_index("c")
    s = lax.axis_index("s")

    pltpu.sync_copy(ids_hbm.at[s], ids_v)
    for j in range(_TPT // 16):
        val_v[j // (_CH // 16), pl.ds((j % (_CH // 16)) * 16, 16)] = (
            s * _TPT + j * 16 + lax.iota(jnp.int32, 16)
        )
    cps = [
        pltpu.async_copy(val_v.at[k], tbl.at[ids_v.at[k]], sem)
        for k in range(_NCH)
    ]
    for cp in cps:
        cp.wait()
    plsc.subcore_barrier()

    def _round(_, carry):
        gs = [
            pltpu.async_copy(tbl.at[ids_v.at[k]], w_v.at[k], sem)
            for k in range(_NCH)
        ]
        for cp in gs:
            cp.wait()
        for k in range(_NCH):
            for j in range(_CH // 16):
                sl = pl.ds(j * 16, 16)
                wv = w_v[k, sl]
                vv = val_v[k, sl]
                iv = ids_v[k, sl]
                idx_v[k, sl] = jnp.where(wv < vv, iv, vv + _NUM)
        ss = [
            pltpu.async_copy(val_v.at[k], tbl.at[idx_v.at[k]], sem)
            for k in range(_NCH)
        ]
        for cp in ss:
            cp.wait()
        plsc.subcore_barrier()
        return carry

    lax.fori_loop(0, _R, _round, 0)
    gs = [
        pltpu.async_copy(tbl.at[ids_v.at[k]], w_v.at[k], sem)
        for k in range(_NCH)
    ]
    for cp in gs:
        cp.wait()

    pltpu.sync_copy(w_v.at[pl.ds(0, _GCH)], wsh.at[2 * s])
    pltpu.sync_copy(w_v.at[pl.ds(_GCH, _GCH)], wsh.at[2 * s + 1])
    plsc.subcore_barrier()

    wid = c * _NT + s
    pltpu.sync_copy(wsh.at[wid], gidx_v)
    sems = (sem, sem_g)
    cps = [None, None]
    cps[0] = pltpu.async_copy(upd_hbm.at[gidx_v.at[0]], rows_v.at[0], sems[0])
    for k in range(_GCH):
        if k + 1 < _GCH:
            cps[(k + 1) % 2] = pltpu.async_copy(
                upd_hbm.at[gidx_v.at[k + 1]], rows_v.at[(k + 1) % 2],
                sems[(k + 1) % 2])
        cps[k % 2].wait()
        pltpu.sync_copy(rows_v.at[k % 2], out_hbm.at[wid, k])


def _dedup_gather_sc(ids3, updated):
    mesh = plsc.VectorSubcoreMesh(core_axis_name="c", subcore_axis_name="s")
    f = functools.partial(
        pl.kernel,
        out_type=jax.ShapeDtypeStruct((_NW, _GCH, _CH, _D), jnp.float32),
        scratch_types=[
            pltpu.VMEM_SHARED((_NUM + _B,), jnp.int32),
            pltpu.VMEM_SHARED((_NW, _GCH, _CH), jnp.int32),
            pltpu.VMEM((_NCH, _CH), jnp.int32),
            pltpu.VMEM((_NCH, _CH), jnp.int32),
            pltpu.VMEM((_NCH, _CH), jnp.int32),
            pltpu.VMEM((_NCH, _CH), jnp.int32),
            pltpu.VMEM((_GCH, _CH), jnp.int32),
            pltpu.VMEM((2, _CH, _D), jnp.float32),
            pltpu.SemaphoreType.DMA,
            pltpu.SemaphoreType.DMA,
        ],
        mesh=mesh,
    )(_fused_body)
    return f(ids3, updated)


def kernel(node_ids, updated_node_memories, new_times, node_memories,
           node_last_updated_times, ln_weight, ln_bias):
    ids3 = node_ids.astype(jnp.int32).reshape(_NT, _NCH, _CH)
    raw = _dedup_gather_sc(ids3, updated_node_memories)
    return _layer_norm_tc(raw.reshape(_B, _D), ln_weight, ln_bias)


# speedup vs baseline: 1.2412x; 1.2412x over previous
import functools

import jax
import jax.numpy as jnp
from jax import lax
from jax.experimental import pallas as pl
from jax.experimental.pallas import tpu as pltpu
from jax.experimental.pallas import tpu_sc as plsc

_B = 16384
_D = 128
_NUM = 1000000
_R = 4

_NT = 16
_TPT = _B // _NT
_CH = 128
_NCH = _TPT // _CH

_NW = 32
_RPW = _B // _NW
_GCH = _RPW // _CH


def _ln_body(x_ref, g_ref, b_ref, o_ref):
    x = x_ref[...]
    mu = jnp.mean(x, axis=-1, keepdims=True)
    xc = x - mu
    var = jnp.mean(xc * xc, axis=-1, keepdims=True)
    o_ref[...] = xc * lax.rsqrt(var + 1e-5) * g_ref[...] + b_ref[...]


def _layer_norm_tc(x, g, b):
    blk = 1024
    return pl.pallas_call(
        _ln_body,
        grid=(_B // blk,),
        in_specs=[
            pl.BlockSpec((blk, _D), lambda i: (i, 0)),
            pl.BlockSpec((1, _D), lambda i: (0, 0)),
            pl.BlockSpec((1, _D), lambda i: (0, 0)),
        ],
        out_specs=pl.BlockSpec((blk, _D), lambda i: (i, 0)),
        out_shape=jax.ShapeDtypeStruct((_B, _D), jnp.float32),
    )(x, g.reshape(1, _D), b.reshape(1, _D))


def _dedup_body(ids_hbm, w_hbm, tbl, ids_v, val_v, w_v, idx_v, sem):
    c = lax.axis_index("c")
    s = lax.axis_index("s")

    @pl.when(c == 0)
    def _work():
        pltpu.sync_copy(ids_hbm.at[s], ids_v)
        for j in range(_TPT // 16):
            val_v[j // (_CH // 16), pl.ds((j % (_CH // 16)) * 16, 16)] = (
                s * _TPT + j * 16 + lax.iota(jnp.int32, 16)
            )
        cps = [
            pltpu.async_copy(val_v.at[k], tbl.at[ids_v.at[k]], sem)
            for k in range(_NCH)
        ]
        for cp in cps:
            cp.wait()
        plsc.subcore_barrier()

        def _round_full(r, carry):
            gs = [
                pltpu.async_copy(tbl.at[ids_v.at[k]], w_v.at[k], sem)
                for k in range(_NCH)
            ]
            for cp in gs:
                cp.wait()
            for k in range(_NCH):
                for j in range(_CH // 16):
                    sl = pl.ds(j * 16, 16)
                    wv = w_v[k, sl]
                    vv = val_v[k, sl]
                    iv = ids_v[k, sl]
                    idx_v[k, sl] = jnp.where(wv < vv, iv, vv + _NUM)
            ss = [
                pltpu.async_copy(val_v.at[k], tbl.at[idx_v.at[k]], sem)
                for k in range(_NCH)
            ]
            for cp in ss:
                cp.wait()
            plsc.subcore_barrier()
            return carry

        lax.fori_loop(0, _R, _round_full, 0)
        gs = [
            pltpu.async_copy(tbl.at[ids_v.at[k]], w_v.at[k], sem)
            for k in range(_NCH)
        ]
        for cp in gs:
            cp.wait()
        pltpu.sync_copy(w_v, w_hbm.at[s])


def _dedup_sc(ids3):
    mesh = plsc.VectorSubcoreMesh(core_axis_name="c", subcore_axis_name="s")
    f = functools.partial(
        pl.kernel,
        out_type=jax.ShapeDtypeStruct((_NT, _NCH, _CH), jnp.int32),
        scratch_types=[
            pltpu.VMEM_SHARED((_NUM + _B,), jnp.int32),
            pltpu.VMEM((_NCH, _CH), jnp.int32),
            pltpu.VMEM((_NCH, _CH), jnp.int32),
            pltpu.VMEM((_NCH, _CH), jnp.int32),
            pltpu.VMEM((_NCH, _CH), jnp.int32),
            pltpu.SemaphoreType.DMA,
        ],
        mesh=mesh,
    )(_dedup_body)
    return f(ids3)


def _gather_body(norm_hbm, widx_hbm, out_hbm, idx_v, rows_v, sem):
    c = lax.axis_index("c")
    s = lax.axis_index("s")
    wid = s * 2 + c
    pltpu.sync_copy(widx_hbm.at[wid], idx_v)
    cps = [
        pltpu.async_copy(norm_hbm.at[idx_v.at[k]], rows_v.at[k], sem)
        for k in range(_GCH)
    ]
    for cp in cps:
        cp.wait()
    pltpu.sync_copy(rows_v, out_hbm.at[wid])


def _gather_sc(normalized, widx):
    mesh = plsc.VectorSubcoreMesh(core_axis_name="c", subcore_axis_name="s")
    f = functools.partial(
        pl.kernel,
        out_type=jax.ShapeDtypeStruct((_NW, _GCH, _CH, _D), jnp.float32),
        scratch_types=[
            pltpu.VMEM((_GCH, _CH), jnp.int32),
            pltpu.VMEM((_GCH, _CH, _D), jnp.float32),
            pltpu.SemaphoreType.DMA,
        ],
        mesh=mesh,
    )(_gather_body)
    return f(normalized, widx)


def kernel(node_ids, updated_node_memories, new_times, node_memories,
           node_last_updated_times, ln_weight, ln_bias):
    ids3 = node_ids.astype(jnp.int32).reshape(_NT, _NCH, _CH)
    normalized = _layer_norm_tc(updated_node_memories, ln_weight, ln_bias)
    winner = _dedup_sc(ids3)
    widx = winner.reshape(_NW, _GCH, _CH)
    out = _gather_sc(normalized, widx)
    return out.reshape(_B, _D)


# speedup vs baseline: 1.2812x; 1.0322x over previous
import functools

import jax
import jax.numpy as jnp
from jax import lax
from jax.experimental import pallas as pl
from jax.experimental.pallas import tpu as pltpu
from jax.experimental.pallas import tpu_sc as plsc

_B = 16384
_D = 128
_NUM = 1000000
_R = 4

_NT = 16
_TPT = _B // _NT
_CH = 128
_NCH = _TPT // _CH

_NW = 32
_RPW = _B // _NW
_GCH = _RPW // _CH


def _ln_body(x_ref, g_ref, b_ref, o_ref):
    x = x_ref[...]
    mu = jnp.mean(x, axis=-1, keepdims=True)
    xc = x - mu
    var = jnp.mean(xc * xc, axis=-1, keepdims=True)
    o_ref[...] = xc * lax.rsqrt(var + 1e-5) * g_ref[...] + b_ref[...]


def _layer_norm_tc(x, g, b):
    blk = 4096
    return pl.pallas_call(
        _ln_body,
        grid=(_B // blk,),
        in_specs=[
            pl.BlockSpec((blk, _D), lambda i: (i, 0)),
            pl.BlockSpec((1, _D), lambda i: (0, 0)),
            pl.BlockSpec((1, _D), lambda i: (0, 0)),
        ],
        out_specs=pl.BlockSpec((blk, _D), lambda i: (i, 0)),
        out_shape=jax.ShapeDtypeStruct((_B, _D), jnp.float32),
    )(x, g.reshape(1, _D), b.reshape(1, _D))


def _dedup_body(ids_hbm, w_hbm, tbl, ids_v, val_v, w_v, idx_v, sem):
    c = lax.axis_index("c")
    s = lax.axis_index("s")

    @pl.when(c == 0)
    def _work():
        pltpu.sync_copy(ids_hbm.at[s], ids_v)
        for j in range(_TPT // 16):
            val_v[j // (_CH // 16), pl.ds((j % (_CH // 16)) * 16, 16)] = (
                s * _TPT + j * 16 + lax.iota(jnp.int32, 16)
            )
        cps = [
            pltpu.async_copy(val_v.at[k], tbl.at[ids_v.at[k]], sem)
            for k in range(_NCH)
        ]
        for cp in cps:
            cp.wait()
        plsc.subcore_barrier()

        def _round_full(r, carry):
            gs = [
                pltpu.async_copy(tbl.at[ids_v.at[k]], w_v.at[k], sem)
                for k in range(_NCH)
            ]
            for cp in gs:
                cp.wait()
            for k in range(_NCH):
                for j in range(_CH // 16):
                    sl = pl.ds(j * 16, 16)
                    wv = w_v[k, sl]
                    vv = val_v[k, sl]
                    iv = ids_v[k, sl]
                    idx_v[k, sl] = jnp.where(wv < vv, iv, vv + _NUM)
            ss = [
                pltpu.async_copy(val_v.at[k], tbl.at[idx_v.at[k]], sem)
                for k in range(_NCH)
            ]
            for cp in ss:
                cp.wait()
            plsc.subcore_barrier()
            return carry

        lax.fori_loop(0, _R, _round_full, 0)
        gs = [
            pltpu.async_copy(tbl.at[ids_v.at[k]], w_v.at[k], sem)
            for k in range(_NCH)
        ]
        for cp in gs:
            cp.wait()
        pltpu.sync_copy(w_v, w_hbm.at[s])


def _dedup_sc(ids3):
    mesh = plsc.VectorSubcoreMesh(core_axis_name="c", subcore_axis_name="s")
    f = functools.partial(
        pl.kernel,
        out_type=jax.ShapeDtypeStruct((_NT, _NCH, _CH), jnp.int32),
        scratch_types=[
            pltpu.VMEM_SHARED((_NUM + _B,), jnp.int32),
            pltpu.VMEM((_NCH, _CH), jnp.int32),
            pltpu.VMEM((_NCH, _CH), jnp.int32),
            pltpu.VMEM((_NCH, _CH), jnp.int32),
            pltpu.VMEM((_NCH, _CH), jnp.int32),
            pltpu.SemaphoreType.DMA,
        ],
        mesh=mesh,
    )(_dedup_body)
    return f(ids3)


def _gather_body(norm_hbm, widx_hbm, out_hbm, idx_v, rows_v, sem):
    c = lax.axis_index("c")
    s = lax.axis_index("s")
    wid = s * 2 + c
    pltpu.sync_copy(widx_hbm.at[wid], idx_v)
    cps = [
        pltpu.async_copy(norm_hbm.at[idx_v.at[k]], rows_v.at[k], sem)
        for k in range(_GCH)
    ]
    for cp in cps:
        cp.wait()
    pltpu.sync_copy(rows_v, out_hbm.at[wid])


def _gather_sc(normalized, widx):
    mesh = plsc.VectorSubcoreMesh(core_axis_name="c", subcore_axis_name="s")
    f = functools.partial(
        pl.kernel,
        out_type=jax.ShapeDtypeStruct((_NW, _GCH, _CH, _D), jnp.float32),
        scratch_types=[
            pltpu.VMEM((_GCH, _CH), jnp.int32),
            pltpu.VMEM((_GCH, _CH, _D), jnp.float32),
            pltpu.SemaphoreType.DMA,
        ],
        mesh=mesh,
    )(_gather_body)
    return f(normalized, widx)


def kernel(node_ids, updated_node_memories, new_times, node_memories,
           node_last_updated_times, ln_weight, ln_bias):
    ids3 = node_ids.astype(jnp.int32).reshape(_NT, _NCH, _CH)
    normalized = _layer_norm_tc(updated_node_memories, ln_weight, ln_bias)
    winner = _dedup_sc(ids3)
    widx = winner.reshape(_NW, _GCH, _CH)
    out = _gather_sc(normalized, widx)
    return out.reshape(_B, _D)


# speedup vs baseline: 1.3011x; 1.0155x over previous
import functools

import jax
import jax.numpy as jnp
from jax import lax
from jax.experimental import pallas as pl
from jax.experimental.pallas import tpu as pltpu
from jax.experimental.pallas import tpu_sc as plsc

_B = 16384
_D = 128
_NUM = 1000000
_R = 4

_NT = 16
_TPT = _B // _NT
_CH = 128
_NCH = _TPT // _CH

_NW = 32
_RPW = _B // _NW
_GCH = _RPW // _CH


def _ln_body(x_ref, g_ref, b_ref, o_ref):
    x = x_ref[...]
    mu = jnp.mean(x, axis=-1, keepdims=True)
    xc = x - mu
    var = jnp.mean(xc * xc, axis=-1, keepdims=True)
    o_ref[...] = xc * lax.rsqrt(var + 1e-5) * g_ref[...] + b_ref[...]


def _layer_norm_tc(x, g, b):
    blk = 8192
    return pl.pallas_call(
        _ln_body,
        grid=(_B // blk,),
        in_specs=[
            pl.BlockSpec((blk, _D), lambda i: (i, 0)),
            pl.BlockSpec((1, _D), lambda i: (0, 0)),
            pl.BlockSpec((1, _D), lambda i: (0, 0)),
        ],
        out_specs=pl.BlockSpec((blk, _D), lambda i: (i, 0)),
        out_shape=jax.ShapeDtypeStruct((_B, _D), jnp.float32),
    )(x, g.reshape(1, _D), b.reshape(1, _D))


def _dedup_body(ids_hbm, w_hbm, tbl, ids_v, val_v, w_v, idx_v, sem):
    c = lax.axis_index("c")
    s = lax.axis_index("s")

    @pl.when(c == 0)
    def _work():
        pltpu.sync_copy(ids_hbm.at[s], ids_v)
        for j in range(_TPT // 16):
            val_v[j // (_CH // 16), pl.ds((j % (_CH // 16)) * 16, 16)] = (
                s * _TPT + j * 16 + lax.iota(jnp.int32, 16)
            )
        cps = [
            pltpu.async_copy(val_v.at[k], tbl.at[ids_v.at[k]], sem)
            for k in range(_NCH)
        ]
        for cp in cps:
            cp.wait()
        plsc.subcore_barrier()

        def _round_full(r, carry):
            gs = [
                pltpu.async_copy(tbl.at[ids_v.at[k]], w_v.at[k], sem)
                for k in range(_NCH)
            ]
            for cp in gs:
                cp.wait()
            for k in range(_NCH):
                for j in range(_CH // 16):
                    sl = pl.ds(j * 16, 16)
                    wv = w_v[k, sl]
                    vv = val_v[k, sl]
                    iv = ids_v[k, sl]
                    idx_v[k, sl] = jnp.where(wv < vv, iv, vv + _NUM)
            ss = [
                pltpu.async_copy(val_v.at[k], tbl.at[idx_v.at[k]], sem)
                for k in range(_NCH)
            ]
            for cp in ss:
                cp.wait()
            plsc.subcore_barrier()
            return carry

        lax.fori_loop(0, _R, _round_full, 0)
        gs = [
            pltpu.async_copy(tbl.at[ids_v.at[k]], w_v.at[k], sem)
            for k in range(_NCH)
        ]
        for cp in gs:
            cp.wait()
        pltpu.sync_copy(w_v, w_hbm.at[s])


def _dedup_sc(ids3):
    mesh = plsc.VectorSubcoreMesh(core_axis_name="c", subcore_axis_name="s")
    f = functools.partial(
        pl.kernel,
        out_type=jax.ShapeDtypeStruct((_NT, _NCH, _CH), jnp.int32),
        scratch_types=[
            pltpu.VMEM_SHARED((_NUM + _B,), jnp.int32),
            pltpu.VMEM((_NCH, _CH), jnp.int32),
            pltpu.VMEM((_NCH, _CH), jnp.int32),
            pltpu.VMEM((_NCH, _CH), jnp.int32),
            pltpu.VMEM((_NCH, _CH), jnp.int32),
            pltpu.SemaphoreType.DMA,
        ],
        mesh=mesh,
    )(_dedup_body)
    return f(ids3)


def _gather_body(norm_hbm, widx_hbm, out_hbm, idx_v, rows_v, sem):
    c = lax.axis_index("c")
    s = lax.axis_index("s")
    wid = s * 2 + c
    pltpu.sync_copy(widx_hbm.at[wid], idx_v)
    cps = [
        pltpu.async_copy(norm_hbm.at[idx_v.at[k]], rows_v.at[k], sem)
        for k in range(_GCH)
    ]
    for cp in cps:
        cp.wait()
    pltpu.sync_copy(rows_v, out_hbm.at[wid])


def _gather_sc(normalized, widx):
    mesh = plsc.VectorSubcoreMesh(core_axis_name="c", subcore_axis_name="s")
    f = functools.partial(
        pl.kernel,
        out_type=jax.ShapeDtypeStruct((_NW, _GCH, _CH, _D), jnp.float32),
        scratch_types=[
            pltpu.VMEM((_GCH, _CH), jnp.int32),
            pltpu.VMEM((_GCH, _CH, _D), jnp.float32),
            pltpu.SemaphoreType.DMA,
        ],
        mesh=mesh,
    )(_gather_body)
    return f(normalized, widx)


def kernel(node_ids, updated_node_memories, new_times, node_memories,
           node_last_updated_times, ln_weight, ln_bias):
    ids3 = node_ids.astype(jnp.int32).reshape(_NT, _NCH, _CH)
    normalized = _layer_norm_tc(updated_node_memories, ln_weight, ln_bias)
    winner = _dedup_sc(ids3)
    widx = winner.reshape(_NW, _GCH, _CH)
    out = _gather_sc(normalized, widx)
    return out.reshape(_B, _D)
